# initial kernel scaffold (unmeasured)
import jax
import jax.numpy as jnp
from jax import lax
from jax.experimental import pallas as pl
from jax.experimental.pallas import tpu as pltpu

S = 1024
D = 2048
DC_SH = 128
H = 16
DH = 128
DR = 32
SCALE = (DH + DR) ** -0.5
BF16 = jnp.bfloat16


def kernel(x, Wdkv, Wuk, Wuv, Wq, Wqr, Wkr, Wo):
    def body(
        x_ref, wdkv_ref, wuk_ref, wuv_ref, wq_ref, wqr_ref, wkr_ref, wo_ref,
        out_ref,
        xbf, c_loc, c_rem, wuk_bf, wuk_rem, wuv_bf, wuv_rem, o_acc,
        send_sems, recv_sems,
    ):
        my_x = lax.axis_index("x")
        my_y = lax.axis_index("y")
        peer = (1 - my_x, my_y)

        barrier_sem = pltpu.get_barrier_semaphore()
        pl.semaphore_signal(
            barrier_sem, inc=1, device_id=peer,
            device_id_type=pl.DeviceIdType.MESH,
        )
        pl.semaphore_wait(barrier_sem, 1)

        wuk_bf[...] = wuk_ref[...].astype(BF16)
        wuv_bf[...] = wuv_ref[...].astype(BF16)
        rdma_wuk = pltpu.make_async_remote_copy(
            src_ref=wuk_bf, dst_ref=wuk_rem,
            send_sem=send_sems.at[0], recv_sem=recv_sems.at[0],
            device_id=peer, device_id_type=pl.DeviceIdType.MESH,
        )
        rdma_wuk.start()
        rdma_wuv = pltpu.make_async_remote_copy(
            src_ref=wuv_bf, dst_ref=wuv_rem,
            send_sem=send_sems.at[1], recv_sem=recv_sems.at[1],
            device_id=peer, device_id_type=pl.DeviceIdType.MESH,
        )
        rdma_wuv.start()

        xbf[...] = x_ref[0].astype(BF16)
        c_loc[...] = jnp.dot(
            xbf[...], wdkv_ref[...].astype(BF16), preferred_element_type=BF16
        )
        rdma_c = pltpu.make_async_remote_copy(
            src_ref=c_loc, dst_ref=c_rem,
            send_sem=send_sems.at[2], recv_sem=recv_sems.at[2],
            device_id=peer, device_id_type=pl.DeviceIdType.MESH,
        )
        rdma_c.start()

        q = jnp.dot(xbf[...], wq_ref[...].astype(BF16), preferred_element_type=BF16)
        qr = jnp.dot(xbf[...], wqr_ref[...].astype(BF16), preferred_element_type=BF16)
        kr = jnp.dot(xbf[...], wkr_ref[...].astype(BF16), preferred_element_type=BF16)

        rdma_wuk.wait()
        rdma_wuv.wait()
        rdma_c.wait()

        k = (
            jnp.dot(c_loc[...], wuk_bf[...], preferred_element_type=jnp.float32)
            + jnp.dot(c_rem[...], wuk_rem[...], preferred_element_type=jnp.float32)
        ).astype(BF16)
        v = (
            jnp.dot(c_loc[...], wuv_bf[...], preferred_element_type=jnp.float32)
            + jnp.dot(c_rem[...], wuv_rem[...], preferred_element_type=jnp.float32)
        ).astype(BF16)

        for h in range(H):
            qh = q[:, h * DH:(h + 1) * DH]
            kh = k[:, h * DH:(h + 1) * DH]
            vh = v[:, h * DH:(h + 1) * DH]
            qrh = qr[:, h * DR:(h + 1) * DR]
            s = lax.dot_general(
                qh, kh, (((1,), (1,)), ((), ())),
                preferred_element_type=jnp.float32,
            )
            s = s + lax.dot_general(
                qrh, kr, (((1,), (1,)), ((), ())),
                preferred_element_type=jnp.float32,
            )
            s = s * SCALE
            m = jnp.max(s, axis=-1, keepdims=True)
            p = jnp.exp(s - m)
            p = p / jnp.sum(p, axis=-1, keepdims=True)
            o_acc[:, h * DH:(h + 1) * DH] = jnp.dot(
                p.astype(BF16), vh, preferred_element_type=jnp.float32
            ).astype(BF16)

        out_ref[0] = jnp.dot(
            o_acc[...], wo_ref[...].astype(BF16),
            preferred_element_type=jnp.float32,
        )

    return pl.pallas_call(
        body,
        out_shape=jax.ShapeDtypeStruct((1, S, D), jnp.float32),
        in_specs=[pl.BlockSpec(memory_space=pltpu.VMEM)] * 8,
        out_specs=pl.BlockSpec(memory_space=pltpu.VMEM),
        scratch_shapes=[
            pltpu.VMEM((S, D), BF16),
            pltpu.VMEM((S, DC_SH), BF16),
            pltpu.VMEM((S, DC_SH), BF16),
            pltpu.VMEM((DC_SH, D), BF16),
            pltpu.VMEM((DC_SH, D), BF16),
            pltpu.VMEM((DC_SH, D), BF16),
            pltpu.VMEM((DC_SH, D), BF16),
            pltpu.VMEM((S, H * DH), BF16),
            pltpu.SemaphoreType.DMA((3,)),
            pltpu.SemaphoreType.DMA((3,)),
        ],
        compiler_params=pltpu.CompilerParams(collective_id=0),
    )(x, Wdkv, Wuk, Wuv, Wq, Wqr, Wkr, Wo)


# baseline (device time: 122826 ns/iter reference)
import jax
import jax.numpy as jnp
from jax import lax
from jax.experimental import pallas as pl
from jax.experimental.pallas import tpu as pltpu

S = 1024
D = 2048
DC_SH = 128
H = 16
DH = 128
DR = 32
SCALE = (DH + DR) ** -0.5
BF16 = jnp.bfloat16
F32 = jnp.float32
CK = 512


def kernel(x, Wdkv, Wuk, Wuv, Wq, Wqr, Wkr, Wo):
    def body(
        x_ref, wdkv_ref, wuk_ref, wuv_ref, wq_ref, wqr_ref, wkr_ref, wo_ref,
        out_ref,
        c_loc, c_rem, wuk_rem, wuv_rem, q_buf, k_buf, v_buf, o_acc, s_buf,
        send_sems, recv_sems,
    ):
        my_x = lax.axis_index("x")
        my_y = lax.axis_index("y")
        peer = (1 - my_x, my_y)

        barrier_sem = pltpu.get_barrier_semaphore()
        pl.semaphore_signal(
            barrier_sem, inc=1, device_id=peer,
            device_id_type=pl.DeviceIdType.MESH,
        )
        pl.semaphore_wait(barrier_sem, 1)

        rdma_wuk = pltpu.make_async_remote_copy(
            src_ref=wuk_ref, dst_ref=wuk_rem,
            send_sem=send_sems.at[0], recv_sem=recv_sems.at[0],
            device_id=peer, device_id_type=pl.DeviceIdType.MESH,
        )
        rdma_wuk.start()
        rdma_wuv = pltpu.make_async_remote_copy(
            src_ref=wuv_ref, dst_ref=wuv_rem,
            send_sem=send_sems.at[1], recv_sem=recv_sems.at[1],
            device_id=peer, device_id_type=pl.DeviceIdType.MESH,
        )
        rdma_wuv.start()

        xb = x_ref[0]
        c_loc[...] = jnp.dot(
            xb, wdkv_ref[...], preferred_element_type=F32
        ).astype(BF16)
        rdma_c = pltpu.make_async_remote_copy(
            src_ref=c_loc, dst_ref=c_rem,
            send_sem=send_sems.at[2], recv_sem=recv_sems.at[2],
            device_id=peer, device_id_type=pl.DeviceIdType.MESH,
        )
        rdma_c.start()

        for j in range(0, D, CK):
            q_buf[:, j:j + CK] = jnp.dot(
                xb, wq_ref[:, j:j + CK], preferred_element_type=F32
            ).astype(BF16)
        qr = jnp.dot(xb, wqr_ref[...], preferred_element_type=F32).astype(BF16)
        kr = jnp.dot(xb, wkr_ref[...], preferred_element_type=F32).astype(BF16)

        rdma_wuk.wait()
        rdma_wuv.wait()
        rdma_c.wait()

        for j in range(0, D, CK):
            k_buf[:, j:j + CK] = (
                jnp.dot(c_loc[...], wuk_ref[:, j:j + CK],
                        preferred_element_type=F32)
                + jnp.dot(c_rem[...], wuk_rem[:, j:j + CK],
                          preferred_element_type=F32)
            ).astype(BF16)
            v_buf[:, j:j + CK] = (
                jnp.dot(c_loc[...], wuv_ref[:, j:j + CK],
                        preferred_element_type=F32)
                + jnp.dot(c_rem[...], wuv_rem[:, j:j + CK],
                          preferred_element_type=F32)
            ).astype(BF16)

        for h in range(H):
            qh = q_buf[:, h * DH:(h + 1) * DH]
            kh = k_buf[:, h * DH:(h + 1) * DH]
            vh = v_buf[:, h * DH:(h + 1) * DH]
            qrh = qr[:, h * DR:(h + 1) * DR]
            s_buf[...] = lax.dot_general(
                qh, kh, (((1,), (1,)), ((), ())),
                preferred_element_type=F32,
            ) + lax.dot_general(
                qrh, kr, (((1,), (1,)), ((), ())),
                preferred_element_type=F32,
            )
            m = jnp.max(s_buf[...], axis=-1, keepdims=True)
            s_buf[...] = jnp.exp(s_buf[...] * SCALE - m * SCALE)
            denom = jnp.sum(s_buf[...], axis=-1, keepdims=True)
            p = (s_buf[...] / denom).astype(BF16)
            o_acc[:, h * DH:(h + 1) * DH] = jnp.dot(
                p, vh, preferred_element_type=F32
            ).astype(BF16)

        for j in range(0, D, CK):
            out_ref[0, :, j:j + CK] = jnp.dot(
                o_acc[...], wo_ref[:, j:j + CK], preferred_element_type=F32
            )

    call = pl.pallas_call(
        body,
        out_shape=jax.ShapeDtypeStruct((1, S, D), F32),
        in_specs=[pl.BlockSpec(memory_space=pltpu.VMEM)] * 8,
        out_specs=pl.BlockSpec(memory_space=pltpu.VMEM),
        scratch_shapes=[
            pltpu.VMEM((S, DC_SH), BF16),
            pltpu.VMEM((S, DC_SH), BF16),
            pltpu.VMEM((DC_SH, D), BF16),
            pltpu.VMEM((DC_SH, D), BF16),
            pltpu.VMEM((S, D), BF16),
            pltpu.VMEM((S, D), BF16),
            pltpu.VMEM((S, D), BF16),
            pltpu.VMEM((S, H * DH), BF16),
            pltpu.VMEM((S, S), F32),
            pltpu.SemaphoreType.DMA((3,)),
            pltpu.SemaphoreType.DMA((3,)),
        ],
        compiler_params=pltpu.CompilerParams(
            collective_id=0,
            vmem_limit_bytes=63 * 1024 * 1024,
        ),
    )
    return call(
        x.astype(BF16), Wdkv.astype(BF16), Wuk.astype(BF16), Wuv.astype(BF16),
        Wq.astype(BF16), Wqr.astype(BF16), Wkr.astype(BF16), Wo.astype(BF16),
    )


# device time: 122148 ns/iter; 1.0056x vs baseline; 1.0056x over previous
import jax
import jax.numpy as jnp
from jax import lax
from jax.experimental import pallas as pl
from jax.experimental.pallas import tpu as pltpu

S = 1024
D = 2048
DC_SH = 128
H = 16
HH = 8
DH = 128
DR = 32
HD2 = HH * DH
SCALE = (DH + DR) ** -0.5
BF16 = jnp.bfloat16
F32 = jnp.float32
CK = 512


def kernel(x, Wdkv, Wuk, Wuv, Wq, Wqr, Wkr, Wo):
    def body(
        x_ref, wdkv_ref, wuk_ref, wuv_ref, wq_ref, wqr_ref, wkr_ref, wo_ref,
        out_ref,
        c_loc, c_rem, wuk_snd, wuk_rem, wuv_snd, wuv_rem,
        q_buf, k_buf, v_buf, o_acc, o_rem, s_buf,
        xsend_sems, xrecv_sems, osend_sems, orecv_sems,
    ):
        my_x = lax.axis_index("x")
        my_y = lax.axis_index("y")
        xpeer = (1 - my_x, my_y)
        ypeer = (my_x, 1 - my_y)

        barrier_sem = pltpu.get_barrier_semaphore()
        for nbr in (xpeer, ypeer):
            pl.semaphore_signal(
                barrier_sem, inc=1, device_id=nbr,
                device_id_type=pl.DeviceIdType.MESH,
            )
        pl.semaphore_wait(barrier_sem, 2)

        def run(yd, yr):
            pd = HD2 - yd

            wuk_snd[...] = wuk_ref[:, yd:yd + HD2]
            wuv_snd[...] = wuv_ref[:, yd:yd + HD2]
            rdma_wuk = pltpu.make_async_remote_copy(
                src_ref=wuk_snd, dst_ref=wuk_rem,
                send_sem=xsend_sems.at[0], recv_sem=xrecv_sems.at[0],
                device_id=xpeer, device_id_type=pl.DeviceIdType.MESH,
            )
            rdma_wuk.start()
            rdma_wuv = pltpu.make_async_remote_copy(
                src_ref=wuv_snd, dst_ref=wuv_rem,
                send_sem=xsend_sems.at[1], recv_sem=xrecv_sems.at[1],
                device_id=xpeer, device_id_type=pl.DeviceIdType.MESH,
            )
            rdma_wuv.start()

            xb = x_ref[0]
            c_loc[...] = jnp.dot(
                xb, wdkv_ref[...], preferred_element_type=F32
            ).astype(BF16)
            rdma_c = pltpu.make_async_remote_copy(
                src_ref=c_loc, dst_ref=c_rem,
                send_sem=xsend_sems.at[2], recv_sem=xrecv_sems.at[2],
                device_id=xpeer, device_id_type=pl.DeviceIdType.MESH,
            )
            rdma_c.start()

            for j in range(0, HD2, CK):
                q_buf[:, j:j + CK] = jnp.dot(
                    xb, wq_ref[:, yd + j:yd + j + CK],
                    preferred_element_type=F32,
                ).astype(BF16)
            qr = jnp.dot(
                xb, wqr_ref[:, yr:yr + HH * DR], preferred_element_type=F32
            ).astype(BF16)
            kr = jnp.dot(
                xb, wkr_ref[...], preferred_element_type=F32
            ).astype(BF16)

            rdma_wuk.wait()
            rdma_wuv.wait()
            rdma_c.wait()

            for j in range(0, HD2, CK):
                k_buf[:, j:j + CK] = (
                    jnp.dot(c_loc[...], wuk_snd[:, j:j + CK],
                            preferred_element_type=F32)
                    + jnp.dot(c_rem[...], wuk_rem[:, j:j + CK],
                              preferred_element_type=F32)
                ).astype(BF16)
                v_buf[:, j:j + CK] = (
                    jnp.dot(c_loc[...], wuv_snd[:, j:j + CK],
                            preferred_element_type=F32)
                    + jnp.dot(c_rem[...], wuv_rem[:, j:j + CK],
                              preferred_element_type=F32)
                ).astype(BF16)

            o_rdmas = []
            for h in range(HH):
                qh = q_buf[:, h * DH:(h + 1) * DH]
                kh = k_buf[:, h * DH:(h + 1) * DH]
                vh = v_buf[:, h * DH:(h + 1) * DH]
                qrh = qr[:, h * DR:(h + 1) * DR]
                s_buf[...] = lax.dot_general(
                    qh, kh, (((1,), (1,)), ((), ())),
                    preferred_element_type=F32,
                )
                s_buf[...] += lax.dot_general(
                    qrh, kr, (((1,), (1,)), ((), ())),
                    preferred_element_type=F32,
                )
                m = jnp.max(s_buf[...], axis=-1, keepdims=True)
                e = jnp.exp(s_buf[...] * SCALE - m * SCALE)
                denom = jnp.sum(e, axis=-1, keepdims=True)
                o = jnp.dot(e.astype(BF16), vh, preferred_element_type=F32)
                o_acc[:, h * DH:(h + 1) * DH] = (o / denom).astype(BF16)
                rdma_o = pltpu.make_async_remote_copy(
                    src_ref=o_acc.at[:, h * DH:(h + 1) * DH],
                    dst_ref=o_rem.at[:, h * DH:(h + 1) * DH],
                    send_sem=osend_sems.at[h], recv_sem=orecv_sems.at[h],
                    device_id=ypeer, device_id_type=pl.DeviceIdType.MESH,
                )
                rdma_o.start()
                o_rdmas.append(rdma_o)

            for j in range(0, D, CK):
                out_ref[0, :, j:j + CK] = jnp.dot(
                    o_acc[...], wo_ref[yd:yd + HD2, j:j + CK],
                    preferred_element_type=F32,
                )
            for rdma_o in o_rdmas:
                rdma_o.wait()
            for j in range(0, D, CK):
                out_ref[0, :, j:j + CK] += jnp.dot(
                    o_rem[...], wo_ref[pd:pd + HD2, j:j + CK],
                    preferred_element_type=F32,
                )

        pl.when(my_y == 0)(lambda: run(0, 0))
        pl.when(my_y == 1)(lambda: run(HD2, HH * DR))

    call = pl.pallas_call(
        body,
        out_shape=jax.ShapeDtypeStruct((1, S, D), F32),
        in_specs=[pl.BlockSpec(memory_space=pltpu.VMEM)] * 8,
        out_specs=pl.BlockSpec(memory_space=pltpu.VMEM),
        scratch_shapes=[
            pltpu.VMEM((S, DC_SH), BF16),
            pltpu.VMEM((S, DC_SH), BF16),
            pltpu.VMEM((DC_SH, HD2), BF16),
            pltpu.VMEM((DC_SH, HD2), BF16),
            pltpu.VMEM((DC_SH, HD2), BF16),
            pltpu.VMEM((DC_SH, HD2), BF16),
            pltpu.VMEM((S, HD2), BF16),
            pltpu.VMEM((S, HD2), BF16),
            pltpu.VMEM((S, HD2), BF16),
            pltpu.VMEM((S, HD2), BF16),
            pltpu.VMEM((S, HD2), BF16),
            pltpu.VMEM((S, S), F32),
            pltpu.SemaphoreType.DMA((3,)),
            pltpu.SemaphoreType.DMA((3,)),
            pltpu.SemaphoreType.DMA((HH,)),
            pltpu.SemaphoreType.DMA((HH,)),
        ],
        compiler_params=pltpu.CompilerParams(
            collective_id=0,
            vmem_limit_bytes=63 * 1024 * 1024,
        ),
    )
    return call(
        x.astype(BF16), Wdkv.astype(BF16), Wuk.astype(BF16), Wuv.astype(BF16),
        Wq.astype(BF16), Wqr.astype(BF16), Wkr.astype(BF16), Wo.astype(BF16),
    )


# device time: 92219 ns/iter; 1.3319x vs baseline; 1.3245x over previous
import jax
import jax.numpy as jnp
from jax import lax
from jax.experimental import pallas as pl
from jax.experimental.pallas import tpu as pltpu

S = 1024
SQ = 512
D = 2048
DC_SH = 128
H = 16
HH = 8
DH = 128
DR = 32
HD2 = HH * DH
SCALE = (DH + DR) ** -0.5
BF16 = jnp.bfloat16
F32 = jnp.float32
CK = 512


def kernel(x, Wdkv, Wuk, Wuv, Wq, Wqr, Wkr, Wo):
    def body(
        x_ref, wdkv_ref, wuk_ref, wuv_ref, wq_ref, wqr_ref, wkr_ref, wo_ref,
        out_ref,
        c_loc, c_rem, wuk_snd, wuk_rem, wuv_snd, wuv_rem,
        q_buf, k_buf, v_buf, o_all, o_snd, o_rcv, s_buf, wstage, xb_buf,
        xsend_sems, xrecv_sems, osend_sems, orecv_sems, cp_sems,
        gsend_sems, grecv_sems,
    ):
        my_x = lax.axis_index("x")
        my_y = lax.axis_index("y")
        xpeer = (1 - my_x, my_y)
        ypeer = (my_x, 1 - my_y)
        dpeer = (1 - my_x, 1 - my_y)
        xrow = my_x * SQ

        barrier_sem = pltpu.get_barrier_semaphore()
        for nbr in (xpeer, ypeer, dpeer):
            pl.semaphore_signal(
                barrier_sem, inc=1, device_id=nbr,
                device_id_type=pl.DeviceIdType.MESH,
            )
        pl.semaphore_wait(barrier_sem, 3)

        def run(yd, yr):
            pd = HD2 - yd

            wuk_snd[...] = wuk_ref[:, yd:yd + HD2].astype(BF16)
            wuv_snd[...] = wuv_ref[:, yd:yd + HD2].astype(BF16)
            rdma_wuk = pltpu.make_async_remote_copy(
                src_ref=wuk_snd, dst_ref=wuk_rem,
                send_sem=xsend_sems.at[0], recv_sem=xrecv_sems.at[0],
                device_id=xpeer, device_id_type=pl.DeviceIdType.MESH,
            )
            rdma_wuk.start()
            rdma_wuv = pltpu.make_async_remote_copy(
                src_ref=wuv_snd, dst_ref=wuv_rem,
                send_sem=xsend_sems.at[1], recv_sem=xrecv_sems.at[1],
                device_id=xpeer, device_id_type=pl.DeviceIdType.MESH,
            )
            rdma_wuv.start()

            xb_buf[...] = x_ref[0].astype(BF16)
            xb = xb_buf[...]
            c_loc[...] = jnp.dot(
                xb, wdkv_ref[...].astype(BF16), preferred_element_type=F32
            ).astype(BF16)
            rdma_c = pltpu.make_async_remote_copy(
                src_ref=c_loc, dst_ref=c_rem,
                send_sem=xsend_sems.at[2], recv_sem=xrecv_sems.at[2],
                device_id=xpeer, device_id_type=pl.DeviceIdType.MESH,
            )
            rdma_c.start()

            xq = xb_buf[pl.ds(xrow, SQ), :]

            def copy_wq(j, slot):
                return pltpu.make_async_copy(
                    wq_ref.at[:, pl.ds(yd + j, CK)],
                    wstage.at[slot], cp_sems.at[slot],
                )

            wqr_cp = pltpu.make_async_copy(
                wqr_ref, wstage.at[0], cp_sems.at[0]
            )
            copy_wq(0, 0).start()
            NQ = HD2 // CK
            for jidx in range(NQ):
                j = jidx * CK
                slot = jidx % 2
                if jidx + 1 < NQ:
                    copy_wq(j + CK, 1 - slot).start()
                else:
                    wqr_cp.start()
                copy_wq(j, slot).wait()
                q_buf[:, j:j + CK] = jnp.dot(
                    xq, wstage[slot].astype(BF16),
                    preferred_element_type=F32,
                ).astype(BF16)
            wqr_cp.wait()
            qr = jnp.dot(
                xq, wstage[0, :, yr:yr + HH * DR].astype(BF16),
                preferred_element_type=F32,
            ).astype(BF16)
            kr = jnp.dot(
                xb, wkr_ref[...].astype(BF16), preferred_element_type=F32
            ).astype(BF16)

            rdma_wuk.wait()
            rdma_wuv.wait()
            rdma_c.wait()

            for j in range(0, HD2, CK):
                k_buf[:, j:j + CK] = (
                    jnp.dot(c_loc[...], wuk_snd[:, j:j + CK],
                            preferred_element_type=F32)
                    + jnp.dot(c_rem[...], wuk_rem[:, j:j + CK],
                              preferred_element_type=F32)
                ).astype(BF16)
                v_buf[:, j:j + CK] = (
                    jnp.dot(c_loc[...], wuv_snd[:, j:j + CK],
                            preferred_element_type=F32)
                    + jnp.dot(c_rem[...], wuv_rem[:, j:j + CK],
                              preferred_element_type=F32)
                ).astype(BF16)

            o_rdmas = []
            for h in range(HH):
                qh = q_buf[:, h * DH:(h + 1) * DH]
                kh = k_buf[:, h * DH:(h + 1) * DH]
                vh = v_buf[:, h * DH:(h + 1) * DH]
                qrh = qr[:, h * DR:(h + 1) * DR]
                s_buf[...] = lax.dot_general(
                    qh, kh, (((1,), (1,)), ((), ())),
                    preferred_element_type=F32,
                )
                s_buf[...] += lax.dot_general(
                    qrh, kr, (((1,), (1,)), ((), ())),
                    preferred_element_type=F32,
                )
                m = jnp.max(s_buf[...], axis=-1, keepdims=True)
                e = jnp.exp(s_buf[...] * SCALE - m * SCALE)
                denom = jnp.sum(e, axis=-1, keepdims=True)
                o = jnp.dot(e.astype(BF16), vh, preferred_element_type=F32)
                ob = (o / denom).astype(BF16)
                o_all[:, yd + h * DH:yd + (h + 1) * DH] = ob
                o_snd[h] = ob
                rdma_o = pltpu.make_async_remote_copy(
                    src_ref=o_snd.at[h], dst_ref=o_rcv.at[h],
                    send_sem=osend_sems.at[h], recv_sem=orecv_sems.at[h],
                    device_id=ypeer, device_id_type=pl.DeviceIdType.MESH,
                )
                rdma_o.start()
                o_rdmas.append(rdma_o)

            def copy_wo(j, slot):
                return pltpu.make_async_copy(
                    wo_ref.at[:, pl.ds(j, CK)],
                    wstage.at[slot], cp_sems.at[slot],
                )

            copy_wo(0, 0).start()
            copy_wo(CK, 1).start()

            for rdma_o in o_rdmas:
                rdma_o.wait()
            for h in range(HH):
                o_all[:, pd + h * DH:pd + (h + 1) * DH] = o_rcv[h]

            NJ = D // CK
            g_rdmas = []
            for jidx in range(NJ):
                j = jidx * CK
                slot = jidx % 2
                copy_wo(j, slot).wait()
                out_ref[0, pl.ds(xrow, SQ), j:j + CK] = jnp.dot(
                    o_all[...], wstage[slot].astype(BF16),
                    preferred_element_type=F32,
                ).astype(BF16)
                if jidx + 2 < NJ:
                    copy_wo(j + 2 * CK, slot).start()
                tgt = xpeer if jidx < NJ // 2 else dpeer
                rdma_g = pltpu.make_async_remote_copy(
                    src_ref=out_ref.at[0, pl.ds(xrow, SQ), pl.ds(j, CK)],
                    dst_ref=out_ref.at[0, pl.ds(xrow, SQ), pl.ds(j, CK)],
                    send_sem=gsend_sems.at[jidx], recv_sem=grecv_sems.at[jidx],
                    device_id=tgt, device_id_type=pl.DeviceIdType.MESH,
                )
                rdma_g.start()
                g_rdmas.append(rdma_g)
            for rdma_g in g_rdmas:
                rdma_g.wait()

        pl.when(my_y == 0)(lambda: run(0, 0))
        pl.when(my_y == 1)(lambda: run(HD2, HH * DR))

    call = pl.pallas_call(
        body,
        out_shape=jax.ShapeDtypeStruct((1, S, D), BF16),
        in_specs=[
            pl.BlockSpec(memory_space=pltpu.VMEM),
            pl.BlockSpec(memory_space=pltpu.VMEM),
            pl.BlockSpec(memory_space=pltpu.VMEM),
            pl.BlockSpec(memory_space=pltpu.VMEM),
            pl.BlockSpec(memory_space=pltpu.MemorySpace.HBM),
            pl.BlockSpec(memory_space=pltpu.MemorySpace.HBM),
            pl.BlockSpec(memory_space=pltpu.VMEM),
            pl.BlockSpec(memory_space=pltpu.MemorySpace.HBM),
        ],
        out_specs=pl.BlockSpec(memory_space=pltpu.VMEM),
        scratch_shapes=[
            pltpu.VMEM((S, DC_SH), BF16),
            pltpu.VMEM((S, DC_SH), BF16),
            pltpu.VMEM((DC_SH, HD2), BF16),
            pltpu.VMEM((DC_SH, HD2), BF16),
            pltpu.VMEM((DC_SH, HD2), BF16),
            pltpu.VMEM((DC_SH, HD2), BF16),
            pltpu.VMEM((SQ, HD2), BF16),
            pltpu.VMEM((S, HD2), BF16),
            pltpu.VMEM((S, HD2), BF16),
            pltpu.VMEM((SQ, D), BF16),
            pltpu.VMEM((HH, SQ, DH), BF16),
            pltpu.VMEM((HH, SQ, DH), BF16),
            pltpu.VMEM((SQ, S), F32),
            pltpu.VMEM((2, D, CK), F32),
            pltpu.VMEM((S, D), BF16),
            pltpu.SemaphoreType.DMA((3,)),
            pltpu.SemaphoreType.DMA((3,)),
            pltpu.SemaphoreType.DMA((HH,)),
            pltpu.SemaphoreType.DMA((HH,)),
            pltpu.SemaphoreType.DMA((2,)),
            pltpu.SemaphoreType.DMA((4,)),
            pltpu.SemaphoreType.DMA((4,)),
        ],
        compiler_params=pltpu.CompilerParams(
            collective_id=0,
            vmem_limit_bytes=63 * 1024 * 1024,
        ),
    )
    return call(x, Wdkv, Wuk, Wuv, Wq, Wqr, Wkr, Wo)
